# baseline (speedup 1.0000x reference)
import functools
import math

import jax
import jax.numpy as jnp
from jax import lax
from jax.experimental import pallas as pl
from jax.experimental.pallas import tpu as pltpu
from jax.experimental.pallas import tpu_sc as plsc

N = 10000
E = 320000
D_IN = 128
H = 64
C_OUT = 16
L = 8
ALPHA = 0.1
THETA = 0.5

NC = 2
NS = 16
NW = NC * NS
K = 128
RB = 2
E_PAD = ((E + NW * K * RB - 1) // (NW * K * RB)) * (NW * K * RB)
EPW = E_PAD // NW
CH = EPW // K
N_PAD = 10240
RPW = N_PAD // NS

BLK = 2000


_sc_mesh = plsc.VectorSubcoreMesh(core_axis_name="c", subcore_axis_name="s")


@functools.partial(
    pl.kernel,
    out_type=jax.ShapeDtypeStruct((NC * N_PAD, H), jnp.float32),
    mesh=_sc_mesh,
    compiler_params=pltpu.CompilerParams(use_tc_tiling_on_sc=False),
    scratch_types=[
        pltpu.VMEM((CH // RB, RB * K), jnp.int32),
        pltpu.VMEM((CH // RB, RB * K), jnp.int32),
        pltpu.VMEM((RB * K, H), jnp.float32),
        pltpu.VMEM_SHARED((N_PAD, H), jnp.float32),
        pltpu.SemaphoreType.DMA,
        pltpu.SemaphoreType.DMA,
    ],
)
def _sc_segment_sum(h_hbm, src_hbm, dst_hbm, out_hbm,
                    src_v, dst_v, buf0, agg_sh, semg, sems):
    c = lax.axis_index("c")
    s = lax.axis_index("s")
    w = s * NC + c

    zv = jnp.zeros((16,), jnp.float32)

    def zrow(r, carry):
        for q in range(H // 16):
            buf0[r, pl.ds(q * 16, 16)] = zv
        return carry

    lax.fori_loop(0, RPW, zrow, 0)
    pltpu.sync_copy(buf0.at[pl.ds(0, RPW)], agg_sh.at[pl.ds(s * RPW, RPW)])
    pltpu.sync_copy(src_hbm.at[w], src_v)
    pltpu.sync_copy(dst_hbm.at[w], dst_v)
    plsc.subcore_barrier()

    def step(t, carry):
        pltpu.async_copy(
            h_hbm.at[src_v.at[t]], buf0, semg).wait()
        pltpu.async_copy(
            buf0, agg_sh.at[dst_v.at[t]], sems, add=True).wait()
        return carry

    lax.fori_loop(0, CH // RB, step, 0)
    plsc.subcore_barrier()

    pltpu.sync_copy(agg_sh.at[pl.ds(s * RPW, RPW)],
                    out_hbm.at[pl.ds(c * N_PAD + s * RPW, RPW)])


def _uaf(x, Au, Bu, Cu, Du, Eu):
    P1 = Au * (x + Bu) + jnp.clip(Cu * jnp.square(x), -100.0, 100.0)
    P2 = Du * (x - Bu)
    P3 = jax.nn.relu(P1) + jnp.log1p(jnp.exp(-jnp.abs(P1)))
    P4 = jax.nn.relu(P2) + jnp.log1p(jnp.exp(-jnp.abs(P2)))
    return P3 - P4 + Eu


def _params(p_ref):
    return (p_ref[0], p_ref[1], p_ref[2], p_ref[3], p_ref[4])


def _tc_pre_body(p_ref, x_ref, w_ref, b_ref, o_ref):
    h = jnp.dot(x_ref[...], w_ref[...], preferred_element_type=jnp.float32)
    o_ref[...] = _uaf(h + b_ref[...], *_params(p_ref))


def _tc_layer_body(p_ref, a0_ref, a1_ref, h0_ref, w_ref, o_ref, *, beta):
    s = (a0_ref[0] + a1_ref[0]) * (1.0 - ALPHA) + ALPHA * h0_ref[...]
    t = (1.0 - beta) * s + beta * jnp.dot(
        s, w_ref[...], preferred_element_type=jnp.float32)
    o_ref[...] = _uaf(t, *_params(p_ref))


def _tc_post_body(h_ref, w_ref, b_ref, o_ref):
    z = jnp.dot(h_ref[...], w_ref[...], preferred_element_type=jnp.float32)
    z = z + b_ref[...]
    m = jnp.max(z, axis=-1, keepdims=True)
    e = jnp.exp(z - m)
    o_ref[...] = (z - m) - jnp.log(jnp.sum(e, axis=-1, keepdims=True))


_SMEM_SPEC = pl.BlockSpec(memory_space=pltpu.SMEM)


def _tc_pre(params, x, W0, b0):
    return pl.pallas_call(
        _tc_pre_body,
        grid=(N // BLK,),
        in_specs=[
            _SMEM_SPEC,
            pl.BlockSpec((BLK, D_IN), lambda i: (i, 0)),
            pl.BlockSpec((D_IN, H), lambda i: (0, 0)),
            pl.BlockSpec((1, H), lambda i: (0, 0)),
        ],
        out_specs=pl.BlockSpec((BLK, H), lambda i: (i, 0)),
        out_shape=jax.ShapeDtypeStruct((N, H), jnp.float32),
    )(params, x, W0, b0)


def _tc_layer(params, aggs, h0, Wl, beta):
    return pl.pallas_call(
        functools.partial(_tc_layer_body, beta=beta),
        grid=(N // BLK,),
        in_specs=[
            _SMEM_SPEC,
            pl.BlockSpec((1, BLK, H), lambda i: (0, i, 0)),
            pl.BlockSpec((1, BLK, H), lambda i: (1, i, 0)),
            pl.BlockSpec((BLK, H), lambda i: (i, 0)),
            pl.BlockSpec((H, H), lambda i: (0, 0)),
        ],
        out_specs=pl.BlockSpec((BLK, H), lambda i: (i, 0)),
        out_shape=jax.ShapeDtypeStruct((N, H), jnp.float32),
    )(params, aggs, aggs, h0, Wl)


def _tc_post(h, W1, b1):
    return pl.pallas_call(
        _tc_post_body,
        grid=(N // BLK,),
        in_specs=[
            pl.BlockSpec((BLK, H), lambda i: (i, 0)),
            pl.BlockSpec((H, C_OUT), lambda i: (0, 0)),
            pl.BlockSpec((1, C_OUT), lambda i: (0, 0)),
        ],
        out_specs=pl.BlockSpec((BLK, C_OUT), lambda i: (i, 0)),
        out_shape=jax.ShapeDtypeStruct((N, C_OUT), jnp.float32),
    )(h, W1, b1)


def kernel(x, edge_index, W0, b0, convW, W1, b1, Au, Bu, Cu, Du, Eu):
    params = jnp.stack([Au, Bu, Cu, Du, Eu]).astype(jnp.float32)

    pad = E_PAD - E
    src = jnp.concatenate([edge_index[0], jnp.zeros((pad,), jnp.int32)])
    dst = jnp.concatenate([edge_index[1], jnp.full((pad,), N, jnp.int32)])
    src_r = src.reshape(NW, CH // RB, RB * K)
    dst_r = dst.reshape(NW, CH // RB, RB * K)

    h = _tc_pre(params, x, W0, b0.reshape(1, H))
    h0 = h
    for l in range(L):
        aggs = _sc_segment_sum(h, src_r, dst_r).reshape(NC, N_PAD, H)
        beta = math.log(THETA / (l + 1) + 1.0)
        h = _tc_layer(params, aggs, h0, convW[l], beta)
    return _tc_post(h, W1, b1.reshape(1, C_OUT))


# speedup vs baseline: 1.3656x; 1.3656x over previous
import functools
import math

import jax
import jax.numpy as jnp
from jax import lax
from jax.experimental import pallas as pl
from jax.experimental.pallas import tpu as pltpu
from jax.experimental.pallas import tpu_sc as plsc

N = 10000
E = 320000
D_IN = 128
H = 64
C_OUT = 16
L = 8
ALPHA = 0.1
THETA = 0.5

NC = 2
NS = 16
NW = NC * NS
K = 128
RB = 1
E_PAD = ((E + NW * K * RB - 1) // (NW * K * RB)) * (NW * K * RB)
EPW = E_PAD // NW
CH = EPW // K
N_PAD = 10240
RPW = N_PAD // NS

BLK = 2000


_sc_mesh = plsc.VectorSubcoreMesh(core_axis_name="c", subcore_axis_name="s")


@functools.partial(
    pl.kernel,
    out_type=jax.ShapeDtypeStruct((NC * N_PAD, H), jnp.float32),
    mesh=_sc_mesh,
    compiler_params=pltpu.CompilerParams(use_tc_tiling_on_sc=False),
    scratch_types=[
        pltpu.VMEM((CH // RB, RB * K), jnp.int32),
        pltpu.VMEM((CH // RB, RB * K), jnp.int32),
        pltpu.VMEM((RB * K, H), jnp.float32),
        pltpu.VMEM_SHARED((N_PAD, H), jnp.float32),
        pltpu.SemaphoreType.DMA,
        pltpu.SemaphoreType.DMA,
    ],
)
def _sc_segment_sum(h_hbm, src_hbm, dst_hbm, out_hbm,
                    src_v, dst_v, buf0, agg_sh, semg, sems):
    c = lax.axis_index("c")
    s = lax.axis_index("s")
    w = s * NC + c

    zv = jnp.zeros((16,), jnp.float32)

    def zrow(r, carry):
        for q in range(H // 16):
            buf0[r, pl.ds(q * 16, 16)] = zv
        return carry

    lax.fori_loop(0, RPW, zrow, 0)
    pltpu.sync_copy(buf0.at[pl.ds(0, RPW)], agg_sh.at[pl.ds(s * RPW, RPW)])
    pltpu.sync_copy(src_hbm.at[w], src_v)
    pltpu.sync_copy(dst_hbm.at[w], dst_v)
    plsc.subcore_barrier()

    def step(t, carry):
        pltpu.async_copy(
            h_hbm.at[src_v.at[t]], buf0, semg).wait()
        pltpu.async_copy(
            buf0, agg_sh.at[dst_v.at[t]], sems, add=True).wait()
        return carry

    lax.fori_loop(0, CH // RB, step, 0)
    plsc.subcore_barrier()

    pltpu.sync_copy(agg_sh.at[pl.ds(s * RPW, RPW)],
                    out_hbm.at[pl.ds(c * N_PAD + s * RPW, RPW)])


def _uaf(x, Au, Bu, Cu, Du, Eu):
    P1 = Au * (x + Bu) + jnp.clip(Cu * jnp.square(x), -100.0, 100.0)
    P2 = Du * (x - Bu)
    P3 = jax.nn.relu(P1) + jnp.log1p(jnp.exp(-jnp.abs(P1)))
    P4 = jax.nn.relu(P2) + jnp.log1p(jnp.exp(-jnp.abs(P2)))
    return P3 - P4 + Eu


def _params(p_ref):
    return (p_ref[0], p_ref[1], p_ref[2], p_ref[3], p_ref[4])


def _tc_pre_body(p_ref, x_ref, w_ref, b_ref, o_ref):
    h = jnp.dot(x_ref[...], w_ref[...], preferred_element_type=jnp.float32)
    o_ref[...] = _uaf(h + b_ref[...], *_params(p_ref))


def _tc_layer_body(p_ref, a0_ref, a1_ref, h0_ref, w_ref, o_ref, *, beta):
    s = (a0_ref[0] + a1_ref[0]) * (1.0 - ALPHA) + ALPHA * h0_ref[...]
    t = (1.0 - beta) * s + beta * jnp.dot(
        s, w_ref[...], preferred_element_type=jnp.float32)
    o_ref[...] = _uaf(t, *_params(p_ref))


def _tc_post_body(h_ref, w_ref, b_ref, o_ref):
    z = jnp.dot(h_ref[...], w_ref[...], preferred_element_type=jnp.float32)
    z = z + b_ref[...]
    m = jnp.max(z, axis=-1, keepdims=True)
    e = jnp.exp(z - m)
    o_ref[...] = (z - m) - jnp.log(jnp.sum(e, axis=-1, keepdims=True))


_SMEM_SPEC = pl.BlockSpec(memory_space=pltpu.SMEM)


def _tc_pre(params, x, W0, b0):
    return pl.pallas_call(
        _tc_pre_body,
        grid=(N // BLK,),
        in_specs=[
            _SMEM_SPEC,
            pl.BlockSpec((BLK, D_IN), lambda i: (i, 0)),
            pl.BlockSpec((D_IN, H), lambda i: (0, 0)),
            pl.BlockSpec((1, H), lambda i: (0, 0)),
        ],
        out_specs=pl.BlockSpec((BLK, H), lambda i: (i, 0)),
        out_shape=jax.ShapeDtypeStruct((N, H), jnp.float32),
    )(params, x, W0, b0)


def _tc_layer(params, aggs, h0, Wl, beta):
    return pl.pallas_call(
        functools.partial(_tc_layer_body, beta=beta),
        grid=(N // BLK,),
        in_specs=[
            _SMEM_SPEC,
            pl.BlockSpec((1, BLK, H), lambda i: (0, i, 0)),
            pl.BlockSpec((1, BLK, H), lambda i: (1, i, 0)),
            pl.BlockSpec((BLK, H), lambda i: (i, 0)),
            pl.BlockSpec((H, H), lambda i: (0, 0)),
        ],
        out_specs=pl.BlockSpec((BLK, H), lambda i: (i, 0)),
        out_shape=jax.ShapeDtypeStruct((N, H), jnp.float32),
    )(params, aggs, aggs, h0, Wl)


def _tc_post(h, W1, b1):
    return pl.pallas_call(
        _tc_post_body,
        grid=(N // BLK,),
        in_specs=[
            pl.BlockSpec((BLK, H), lambda i: (i, 0)),
            pl.BlockSpec((H, C_OUT), lambda i: (0, 0)),
            pl.BlockSpec((1, C_OUT), lambda i: (0, 0)),
        ],
        out_specs=pl.BlockSpec((BLK, C_OUT), lambda i: (i, 0)),
        out_shape=jax.ShapeDtypeStruct((N, C_OUT), jnp.float32),
    )(h, W1, b1)


def kernel(x, edge_index, W0, b0, convW, W1, b1, Au, Bu, Cu, Du, Eu):
    params = jnp.stack([Au, Bu, Cu, Du, Eu]).astype(jnp.float32)

    pad = E_PAD - E
    src = jnp.concatenate([edge_index[0], jnp.zeros((pad,), jnp.int32)])
    dst = jnp.concatenate([edge_index[1], jnp.full((pad,), N, jnp.int32)])
    src_r = src.reshape(NW, CH // RB, RB * K)
    dst_r = dst.reshape(NW, CH // RB, RB * K)

    h = _tc_pre(params, x, W0, b0.reshape(1, H))
    h0 = h
    for l in range(L):
        aggs = _sc_segment_sum(h, src_r, dst_r).reshape(NC, N_PAD, H)
        beta = math.log(THETA / (l + 1) + 1.0)
        h = _tc_layer(params, aggs, h0, convW[l], beta)
    return _tc_post(h, W1, b1.reshape(1, C_OUT))


# speedup vs baseline: 2.2474x; 1.6457x over previous
import functools
import math

import jax
import jax.numpy as jnp
from jax import lax
from jax.experimental import pallas as pl
from jax.experimental.pallas import tpu as pltpu
from jax.experimental.pallas import tpu_sc as plsc

N = 10000
E = 320000
D_IN = 128
H = 64
C_OUT = 16
L = 8
ALPHA = 0.1
THETA = 0.5

NC = 2
NS = 16
NW = NC * NS
K = 128
RB = 1
E_PAD = ((E + NW * K * RB - 1) // (NW * K * RB)) * (NW * K * RB)
EPW = E_PAD // NW
CH = EPW // K
N_PAD = 10240
RPW = N_PAD // NS

BLK = 2000


_sc_mesh = plsc.VectorSubcoreMesh(core_axis_name="c", subcore_axis_name="s")


@functools.partial(
    pl.kernel,
    out_type=jax.ShapeDtypeStruct((NC * N_PAD, H), jnp.float32),
    mesh=_sc_mesh,
    compiler_params=pltpu.CompilerParams(use_tc_tiling_on_sc=False),
    scratch_types=[
        pltpu.VMEM((CH // RB, RB * K), jnp.int32),
        pltpu.VMEM((CH // RB, RB * K), jnp.int32),
        pltpu.VMEM((RB * K, H), jnp.float32),
        pltpu.VMEM_SHARED((N, H), jnp.float32),
        pltpu.VMEM_SHARED((N_PAD, H), jnp.float32),
        pltpu.SemaphoreType.DMA,
        pltpu.SemaphoreType.DMA,
    ],
)
def _sc_segment_sum(h_hbm, src_hbm, dst_hbm, out_hbm,
                    src_v, dst_v, buf0, h_sh, agg_sh, semg, sems):
    c = lax.axis_index("c")
    s = lax.axis_index("s")
    w = s * NC + c

    zv = jnp.zeros((16,), jnp.float32)

    def zrow(r, carry):
        for q in range(H // 16):
            buf0[r, pl.ds(q * 16, 16)] = zv
        return carry

    lax.fori_loop(0, RB * K, zrow, 0)
    for t in range(RPW // (RB * K)):
        pltpu.sync_copy(buf0, agg_sh.at[pl.ds(s * RPW + t * RB * K, RB * K)])
    pltpu.sync_copy(h_hbm.at[pl.ds(s * (N // NS), N // NS)],
                    h_sh.at[pl.ds(s * (N // NS), N // NS)])
    pltpu.sync_copy(src_hbm.at[w], src_v)
    pltpu.sync_copy(dst_hbm.at[w], dst_v)
    plsc.subcore_barrier()

    def step(t, carry):
        pltpu.async_copy(
            h_sh.at[src_v.at[t]], buf0, semg).wait()
        pltpu.async_copy(
            buf0, agg_sh.at[dst_v.at[t]], sems, add=True).wait()
        return carry

    lax.fori_loop(0, CH // RB, step, 0)
    plsc.subcore_barrier()

    pltpu.sync_copy(agg_sh.at[pl.ds(s * RPW, RPW)],
                    out_hbm.at[pl.ds(c * N_PAD + s * RPW, RPW)])


def _uaf(x, Au, Bu, Cu, Du, Eu):
    P1 = Au * (x + Bu) + jnp.clip(Cu * jnp.square(x), -100.0, 100.0)
    P2 = Du * (x - Bu)
    P3 = jax.nn.relu(P1) + jnp.log1p(jnp.exp(-jnp.abs(P1)))
    P4 = jax.nn.relu(P2) + jnp.log1p(jnp.exp(-jnp.abs(P2)))
    return P3 - P4 + Eu


def _params(p_ref):
    return (p_ref[0], p_ref[1], p_ref[2], p_ref[3], p_ref[4])


def _tc_pre_body(p_ref, x_ref, w_ref, b_ref, o_ref):
    h = jnp.dot(x_ref[...], w_ref[...], preferred_element_type=jnp.float32)
    o_ref[...] = _uaf(h + b_ref[...], *_params(p_ref))


def _tc_layer_body(p_ref, a0_ref, a1_ref, h0_ref, w_ref, o_ref, *, beta):
    s = (a0_ref[0] + a1_ref[0]) * (1.0 - ALPHA) + ALPHA * h0_ref[...]
    t = (1.0 - beta) * s + beta * jnp.dot(
        s, w_ref[...], preferred_element_type=jnp.float32)
    o_ref[...] = _uaf(t, *_params(p_ref))


def _tc_post_body(h_ref, w_ref, b_ref, o_ref):
    z = jnp.dot(h_ref[...], w_ref[...], preferred_element_type=jnp.float32)
    z = z + b_ref[...]
    m = jnp.max(z, axis=-1, keepdims=True)
    e = jnp.exp(z - m)
    o_ref[...] = (z - m) - jnp.log(jnp.sum(e, axis=-1, keepdims=True))


_SMEM_SPEC = pl.BlockSpec(memory_space=pltpu.SMEM)


def _tc_pre(params, x, W0, b0):
    return pl.pallas_call(
        _tc_pre_body,
        grid=(N // BLK,),
        in_specs=[
            _SMEM_SPEC,
            pl.BlockSpec((BLK, D_IN), lambda i: (i, 0)),
            pl.BlockSpec((D_IN, H), lambda i: (0, 0)),
            pl.BlockSpec((1, H), lambda i: (0, 0)),
        ],
        out_specs=pl.BlockSpec((BLK, H), lambda i: (i, 0)),
        out_shape=jax.ShapeDtypeStruct((N, H), jnp.float32),
    )(params, x, W0, b0)


def _tc_layer(params, aggs, h0, Wl, beta):
    return pl.pallas_call(
        functools.partial(_tc_layer_body, beta=beta),
        grid=(N // BLK,),
        in_specs=[
            _SMEM_SPEC,
            pl.BlockSpec((1, BLK, H), lambda i: (0, i, 0)),
            pl.BlockSpec((1, BLK, H), lambda i: (1, i, 0)),
            pl.BlockSpec((BLK, H), lambda i: (i, 0)),
            pl.BlockSpec((H, H), lambda i: (0, 0)),
        ],
        out_specs=pl.BlockSpec((BLK, H), lambda i: (i, 0)),
        out_shape=jax.ShapeDtypeStruct((N, H), jnp.float32),
    )(params, aggs, aggs, h0, Wl)


def _tc_post(h, W1, b1):
    return pl.pallas_call(
        _tc_post_body,
        grid=(N // BLK,),
        in_specs=[
            pl.BlockSpec((BLK, H), lambda i: (i, 0)),
            pl.BlockSpec((H, C_OUT), lambda i: (0, 0)),
            pl.BlockSpec((1, C_OUT), lambda i: (0, 0)),
        ],
        out_specs=pl.BlockSpec((BLK, C_OUT), lambda i: (i, 0)),
        out_shape=jax.ShapeDtypeStruct((N, C_OUT), jnp.float32),
    )(h, W1, b1)


def kernel(x, edge_index, W0, b0, convW, W1, b1, Au, Bu, Cu, Du, Eu):
    params = jnp.stack([Au, Bu, Cu, Du, Eu]).astype(jnp.float32)

    pad = E_PAD - E
    src = jnp.concatenate([edge_index[0], jnp.zeros((pad,), jnp.int32)])
    dst = jnp.concatenate([edge_index[1], jnp.full((pad,), N, jnp.int32)])
    src_r = src.reshape(NW, CH // RB, RB * K)
    dst_r = dst.reshape(NW, CH // RB, RB * K)

    h = _tc_pre(params, x, W0, b0.reshape(1, H))
    h0 = h
    for l in range(L):
        aggs = _sc_segment_sum(h, src_r, dst_r).reshape(NC, N_PAD, H)
        beta = math.log(THETA / (l + 1) + 1.0)
        h = _tc_layer(params, aggs, h0, convW[l], beta)
    return _tc_post(h, W1, b1.reshape(1, C_OUT))
